# Initial kernel scaffold; baseline (speedup 1.0000x reference)
import jax
import jax.numpy as jnp
from jax.experimental import pallas as pl


def kernel(x, edge_index, batch, Wl1, bl1, Wr1, br1, att1, bias1, Wl2, bl2, Wr2, br2, att2, bias2, Wl3, bl3, Wr3, br3, att3, bias3, Wmu, bmu, Wlv, blv, Wd1, bd1, Wd2, bd2, Wd3, bd3):
    raise NotImplementedError("write your pallas kernel here")


# speedup vs baseline: 1.0012x; 1.0012x over previous
import functools

import jax
import jax.numpy as jnp
from jax.experimental import pallas as pl
from jax.experimental.pallas import tpu as pltpu

N = 50000
E = 1600000
F = 11
H = 3
G = 2048
MAXN = 29
HD = 32
LD = 64
OUTDIM = MAXN * F + MAXN * MAXN
OUTPAD = 1280


def _gatv2(x, src, dst, Wl, bl, Wr, br, att, bias, heads, C, n):
    xl = (x @ Wl + bl).reshape(n, heads, C)
    xr = (x @ Wr + br).reshape(n, heads, C)
    m = jax.nn.leaky_relu(xl[src] + xr[dst], 0.2)
    e = (m * att[None, :, :]).sum(-1)
    emax = jax.ops.segment_max(e, dst, num_segments=n)
    ee = jnp.exp(e - emax[dst])
    denom = jax.ops.segment_sum(ee, dst, num_segments=n)
    alpha = ee / (denom[dst] + 1e-16)
    out = jax.ops.segment_sum(alpha[:, :, None] * xl[src], dst, num_segments=n)
    return out.mean(axis=1) + bias


def _decoder_body(hg_ref, wmu_ref, bmu_ref, wlv_ref, blv_ref, wd1_ref, bd1_ref,
                  wd2_ref, bd2_ref, wd3_ref, bd3_ref, mu_ref, lv_ref, out_ref):
    hg = hg_ref[...]
    mu = hg @ wmu_ref[...] + bmu_ref[...]
    mu_ref[...] = mu
    lv_ref[...] = hg @ wlv_ref[...] + blv_ref[...]
    o = jnp.maximum(mu @ wd1_ref[...] + bd1_ref[...], 0.0)
    o = jnp.maximum(o @ wd2_ref[...] + bd2_ref[...], 0.0)
    out_ref[...] = o @ wd3_ref[...] + bd3_ref[...]


def _decoder(hg, Wmu, bmu, Wlv, blv, Wd1, bd1, Wd2, bd2, Wd3, bd3):
    gb = 256
    Wd3p = jnp.zeros((8 * HD, OUTPAD), jnp.float32).at[:, :OUTDIM].set(Wd3)
    bd3p = jnp.zeros((OUTPAD,), jnp.float32).at[:OUTDIM].set(bd3)
    full = lambda *s: pl.BlockSpec(s, lambda i: (0,) * len(s))
    mu, lv, out = pl.pallas_call(
        _decoder_body,
        grid=(G // gb,),
        in_specs=[
            pl.BlockSpec((gb, 4 * HD), lambda i: (i, 0)),
            full(4 * HD, LD), full(LD), full(4 * HD, LD), full(LD),
            full(LD, 4 * HD), full(4 * HD), full(4 * HD, 8 * HD), full(8 * HD),
            full(8 * HD, OUTPAD), full(OUTPAD),
        ],
        out_specs=[
            pl.BlockSpec((gb, LD), lambda i: (i, 0)),
            pl.BlockSpec((gb, LD), lambda i: (i, 0)),
            pl.BlockSpec((gb, OUTPAD), lambda i: (i, 0)),
        ],
        out_shape=[
            jax.ShapeDtypeStruct((G, LD), jnp.float32),
            jax.ShapeDtypeStruct((G, LD), jnp.float32),
            jax.ShapeDtypeStruct((G, OUTPAD), jnp.float32),
        ],
    )(hg, Wmu, bmu, Wlv, blv, Wd1, bd1, Wd2, bd2, Wd3p, bd3p)
    return mu, lv, out[:, :OUTDIM]


def kernel(x, edge_index, batch, Wl1, bl1, Wr1, br1, att1, bias1, Wl2, bl2,
           Wr2, br2, att2, bias2, Wl3, bl3, Wr3, br3, att3, bias3, Wmu, bmu,
           Wlv, blv, Wd1, bd1, Wd2, bd2, Wd3, bd3):
    n = x.shape[0]
    loop = jnp.arange(n)
    src = jnp.concatenate([edge_index[0], loop])
    dst = jnp.concatenate([edge_index[1], loop])
    h = jax.nn.relu(_gatv2(x, src, dst, Wl1, bl1, Wr1, br1, att1, bias1, H, HD, n))
    h = jax.nn.relu(_gatv2(h, src, dst, Wl2, bl2, Wr2, br2, att2, bias2, H, 2 * HD, n))
    h = jax.nn.relu(_gatv2(h, src, dst, Wl3, bl3, Wr3, br3, att3, bias3, H, 4 * HD, n))
    hg = jax.ops.segment_sum(h, batch, num_segments=G)
    mu, logvar, out = _decoder(hg, Wmu, bmu, Wlv, blv, Wd1, bd1, Wd2, bd2, Wd3, bd3)
    x_rec = out[:, :MAXN * 11].reshape(-1, MAXN, 11)
    adj = out[:, MAXN * 11:].reshape(-1, MAXN, MAXN)
    adj_rec = (adj + jnp.transpose(adj, (0, 2, 1))) / 2.0
    return (x_rec, adj_rec, mu, logvar)


# speedup vs baseline: 17.0489x; 17.0290x over previous
import functools

import jax
import jax.numpy as jnp
from jax import lax
from jax.experimental import pallas as pl
from jax.experimental.pallas import tpu as pltpu
from jax.experimental.pallas import tpu_sc as plsc

N = 50000
E = 1600000
F = 11
H = 3
G = 2048
MAXN = 29
HD = 32
LD = 64
OUTDIM = MAXN * F + MAXN * MAXN
OUTPAD = 1280

NW = 32
NPS = 1600
NPAD = NW * NPS
E2 = E + N
EPAD = 512
E2S = E2 + 256

_mesh = plsc.VectorSubcoreMesh(core_axis_name="c", subcore_axis_name="s")


def _hsum(v):
    parts = [v[i] for i in range(16)]
    while len(parts) > 1:
        parts = [parts[i] + parts[i + 1] for i in range(0, len(parts), 2)]
    return parts[0]


def _gat_sc(C, B, CN, K, BDW, HCP):
    HC = H * C

    @functools.partial(
        pl.kernel, mesh=_mesh,
        out_type=jax.ShapeDtypeStruct((NPAD * C,), jnp.float32),
        scratch_types=[
            pltpu.VMEM((B,), jnp.int32),
            pltpu.VMEM((B + 16,), jnp.int32),
            pltpu.VMEM((B, HCP), jnp.float32),
            pltpu.VMEM((B, HCP), jnp.float32),
            pltpu.VMEM((CN * HC,), jnp.float32),
            pltpu.VMEM((CN * 16,), jnp.float32),
            pltpu.VMEM((CN * C,), jnp.float32),
            pltpu.VMEM((HC,), jnp.float32),
            pltpu.VMEM((C,), jnp.float32),
            pltpu.VMEM((BDW,), jnp.int32),
            pltpu.SemaphoreType.DMA,
            pltpu.SemaphoreType.DMA,
        ],
    )
    def ek(xl_hbm, xr_hbm, srcs_hbm, dsts_hbm, att_hbm, bias_hbm, bd_hbm,
           out_hbm, src_v, dst_v, xl_v, xr_v, acc_v, dn_v, outb_v,
           att_v, bias_v, bd_v, sem1, sem2):
        cid = lax.axis_index("c")
        sid = lax.axis_index("s")
        wid = cid * 16 + sid
        pltpu.sync_copy(bd_hbm.at[wid], bd_v)
        pltpu.sync_copy(att_hbm, att_v)
        pltpu.sync_copy(bias_hbm, bias_v)
        iota = lax.iota(jnp.int32, 16)
        zeros16 = jnp.zeros((16,), jnp.float32)

        def chunk(k, kcarry):
            base = wid * NPS + k * CN

            def zb(i, carry):
                acc_v[pl.ds(i * 16, 16)] = zeros16
                return carry

            lax.fori_loop(0, CN * HC // 16, zb, 0)

            def zb2(i, carry):
                dn_v[pl.ds(i * 16, 16)] = zeros16
                return carry

            lax.fori_loop(0, CN, zb2, 0)

            bdw = bd_v[pl.ds(k, 16)]
            lo = bdw[0]
            hi = bdw[1]
            lo8 = (lo // 8) * 8
            nblk = (hi - lo8 + (B - 1)) // B

            def blk(i, carry):
                e0 = lo8 + i * B
                pltpu.sync_copy(srcs_hbm.at[pl.ds(e0, B)], src_v)
                pltpu.sync_copy(dsts_hbm.at[pl.ds(e0, B)],
                                dst_v.at[pl.ds(0, B)])
                cp1 = pltpu.async_copy(xl_hbm.at[src_v], xl_v, sem1)
                cp2 = pltpu.async_copy(
                    xr_hbm.at[dst_v.at[pl.ds(0, B)]], xr_v, sem2)
                cp1.wait()
                cp2.wait()

                def edge(j, ecarry):
                    dr_raw = dst_v[pl.ds(j, 16)][0] - base
                    inb = (dr_raw >= 0) & (dr_raw < CN)
                    mvf = jnp.where(inb, 1.0, 0.0)
                    dr = jnp.clip(dr_raw, 0, CN - 1)
                    dnvec = zeros16
                    for h in range(H):
                        sacc = zeros16
                        xcache = []
                        for c in range(0, C, 16):
                            xlv = xl_v[j, pl.ds(h * C + c, 16)]
                            xrv = xr_v[j, pl.ds(h * C + c, 16)]
                            xcache.append(xlv)
                            v = xlv + xrv
                            lr = jnp.maximum(v, 0.2 * v)
                            sacc = sacc + att_v[pl.ds(h * C + c, 16)] * lr
                        tot = _hsum(sacc)
                        ee = jnp.exp(jnp.full((16,), tot, jnp.float32)) * mvf
                        onehot = (1 - jnp.clip(jnp.abs(iota - h), 0, 1)
                                  ).astype(jnp.float32)
                        dnvec = dnvec + onehot * ee
                        for ci, c in enumerate(range(0, C, 16)):
                            off = dr * HC + h * C + c
                            acc_v[pl.ds(off, 16)] = (
                                acc_v[pl.ds(off, 16)] + xcache[ci] * ee)
                    dn_v[pl.ds(dr * 16, 16)] = (
                        dn_v[pl.ds(dr * 16, 16)] + dnvec)
                    return ecarry

                lax.fori_loop(0, B, edge, 0)
                return carry

            lax.fori_loop(0, nblk, blk, 0)

            def fin(r, carry):
                dnv = dn_v[pl.ds(r * 16, 16)]
                rec = []
                for h in range(H):
                    rec.append(1.0 / (jnp.full((16,), dnv[h]) + 1e-16))
                for c in range(0, C, 16):
                    o = zeros16
                    for h in range(H):
                        o = o + acc_v[pl.ds(r * HC + h * C + c, 16)] * rec[h]
                    o = o * (1.0 / H) + bias_v[pl.ds(c, 16)]
                    o = jnp.maximum(o, 0.0)
                    outb_v[pl.ds(r * C + c, 16)] = o
                return carry

            lax.fori_loop(0, CN, fin, 0)
            pltpu.sync_copy(outb_v, out_hbm.at[pl.ds(base * C, CN * C)])
            return kcarry

        lax.fori_loop(0, K, chunk, 0)

    return ek


@functools.partial(
    pl.kernel, mesh=_mesh,
    out_type=jax.ShapeDtypeStruct((G * 4 * HD,), jnp.float32),
    scratch_types=[
        pltpu.VMEM((80,), jnp.int32),
        pltpu.VMEM((16, 4 * HD), jnp.float32),
        pltpu.VMEM((64 * 4 * HD,), jnp.float32),
    ],
)
def _pool_sc(h_hbm, gb_hbm, hg_hbm, gb_v, rows_v, acc_v):
    cid = lax.axis_index("c")
    sid = lax.axis_index("s")
    wid = cid * 16 + sid
    pltpu.sync_copy(gb_hbm.at[wid], gb_v)
    zeros16 = jnp.zeros((16,), jnp.float32)
    D = 4 * HD

    def zb(i, carry):
        acc_v[pl.ds(i * 16, 16)] = zeros16
        return carry

    lax.fori_loop(0, 64 * D // 16, zb, 0)

    def graph(g, gcarry):
        gw = gb_v[pl.ds(g, 16)]
        lo = gw[0]
        hi = gw[1]
        lo8 = (lo // 8) * 8
        nblk = (hi - lo8 + 15) // 16

        def blk(i, carry):
            e0 = lo8 + i * 16
            pltpu.sync_copy(h_hbm.at[pl.ds(e0, 16)], rows_v)

            def row(r, rcarry):
                ridx = e0 + r
                mf = jnp.where((ridx >= lo) & (ridx < hi), 1.0, 0.0)
                for c in range(0, D, 16):
                    x = rows_v[r, pl.ds(c, 16)]
                    acc_v[pl.ds(g * D + c, 16)] = (
                        acc_v[pl.ds(g * D + c, 16)] + x * mf)
                return rcarry

            lax.fori_loop(0, 16, row, 0)
            return carry

        lax.fori_loop(0, nblk, blk, 0)
        return gcarry

    lax.fori_loop(0, 64, graph, 0)
    pltpu.sync_copy(acc_v, hg_hbm.at[pl.ds(wid * 64 * D, 64 * D)])


def _proj_body(x_ref, wl_ref, bl_ref, wr_ref, br_ref, xl_ref, xr_ref):
    xv = x_ref[...]
    xl_ref[...] = jnp.dot(
        xv, wl_ref[...], preferred_element_type=jnp.float32) + bl_ref[...]
    xr_ref[...] = jnp.dot(
        xv, wr_ref[...], preferred_element_type=jnp.float32) + br_ref[...]


def _proj(x, Wl, bl, Wr, br):
    cin = x.shape[1]
    hc = Wl.shape[1]
    rb = 512
    full = lambda *s: pl.BlockSpec(s, lambda i: (0,) * len(s))
    return pl.pallas_call(
        _proj_body,
        grid=(NPAD // rb,),
        in_specs=[
            pl.BlockSpec((rb, cin), lambda i: (i, 0)),
            full(cin, hc), full(1, hc), full(cin, hc), full(1, hc),
        ],
        out_specs=[
            pl.BlockSpec((rb, hc), lambda i: (i, 0)),
            pl.BlockSpec((rb, hc), lambda i: (i, 0)),
        ],
        out_shape=[
            jax.ShapeDtypeStruct((NPAD, hc), jnp.float32),
            jax.ShapeDtypeStruct((NPAD, hc), jnp.float32),
        ],
    )(x, Wl, bl.reshape(1, hc), Wr, br.reshape(1, hc))


def _decoder_body(hg_ref, wmu_ref, bmu_ref, wlv_ref, blv_ref, wd1_ref, bd1_ref,
                  wd2_ref, bd2_ref, wd3_ref, bd3_ref, mu_ref, lv_ref, out_ref):
    hg = hg_ref[...]
    mu = hg @ wmu_ref[...] + bmu_ref[...]
    mu_ref[...] = mu
    lv_ref[...] = hg @ wlv_ref[...] + blv_ref[...]
    o = jnp.maximum(mu @ wd1_ref[...] + bd1_ref[...], 0.0)
    o = jnp.maximum(o @ wd2_ref[...] + bd2_ref[...], 0.0)
    out_ref[...] = o @ wd3_ref[...] + bd3_ref[...]


def _decoder(hg, Wmu, bmu, Wlv, blv, Wd1, bd1, Wd2, bd2, Wd3, bd3):
    gb = 256
    Wd3p = jnp.zeros((8 * HD, OUTPAD), jnp.float32).at[:, :OUTDIM].set(Wd3)
    bd3p = jnp.zeros((1, OUTPAD), jnp.float32).at[0, :OUTDIM].set(bd3)
    full = lambda *s: pl.BlockSpec(s, lambda i: (0,) * len(s))
    mu, lv, out = pl.pallas_call(
        _decoder_body,
        grid=(G // gb,),
        in_specs=[
            pl.BlockSpec((gb, 4 * HD), lambda i: (i, 0)),
            full(4 * HD, LD), full(1, LD), full(4 * HD, LD), full(1, LD),
            full(LD, 4 * HD), full(1, 4 * HD), full(4 * HD, 8 * HD),
            full(1, 8 * HD), full(8 * HD, OUTPAD), full(1, OUTPAD),
        ],
        out_specs=[
            pl.BlockSpec((gb, LD), lambda i: (i, 0)),
            pl.BlockSpec((gb, LD), lambda i: (i, 0)),
            pl.BlockSpec((gb, OUTPAD), lambda i: (i, 0)),
        ],
        out_shape=[
            jax.ShapeDtypeStruct((G, LD), jnp.float32),
            jax.ShapeDtypeStruct((G, LD), jnp.float32),
            jax.ShapeDtypeStruct((G, OUTPAD), jnp.float32),
        ],
    )(hg, Wmu, bmu.reshape(1, LD), Wlv, blv.reshape(1, LD), Wd1,
      bd1.reshape(1, 4 * HD), Wd2, bd2.reshape(1, 8 * HD), Wd3p, bd3p)
    return mu, lv, out[:, :OUTDIM]


_L1 = _gat_sc(C=HD, B=128, CN=160, K=10, BDW=32, HCP=128)
_L2 = _gat_sc(C=2 * HD, B=96, CN=160, K=10, BDW=32, HCP=256)
_L3 = _gat_sc(C=4 * HD, B=64, CN=80, K=20, BDW=40, HCP=384)


def _bd_rows(dsts_view, cn, k):
    q = jnp.arange(NW * k + 1, dtype=jnp.int32) * cn
    bd = jnp.searchsorted(dsts_view, q, side="left").astype(jnp.int32)
    rows = bd[(jnp.arange(NW)[:, None] * k + jnp.arange(k + 1)[None, :])]
    padw = 32 if k == 10 else 40
    return jnp.pad(rows, ((0, 0), (0, padw - (k + 1))), mode="edge")


def kernel(x, edge_index, batch, Wl1, bl1, Wr1, br1, att1, bias1, Wl2, bl2,
           Wr2, br2, att2, bias2, Wl3, bl3, Wr3, br3, att3, bias3, Wmu, bmu,
           Wlv, blv, Wd1, bd1, Wd2, bd2, Wd3, bd3):
    loop = jnp.arange(N, dtype=jnp.int32)
    src0 = jnp.concatenate([edge_index[0].astype(jnp.int32), loop])
    dst0 = jnp.concatenate([edge_index[1].astype(jnp.int32), loop])
    dsts, srcs = lax.sort_key_val(dst0, src0)
    srcs = jnp.concatenate([srcs, jnp.zeros((EPAD,), jnp.int32)])
    dsts = jnp.concatenate(
        [dsts, jnp.full((EPAD,), NPAD - 8, jnp.int32)])

    bd10 = _bd_rows(dsts[:E2S], 160, 10)
    bd20 = _bd_rows(dsts[:E2S], 80, 20)

    xp = jnp.zeros((NPAD, 16), jnp.float32).at[:N, :F].set(x)
    Wl1p = jnp.zeros((16, 128), jnp.float32).at[:F, :H * HD].set(Wl1)
    Wr1p = jnp.zeros((16, 128), jnp.float32).at[:F, :H * HD].set(Wr1)
    bl1p = jnp.zeros((128,), jnp.float32).at[:H * HD].set(bl1)
    br1p = jnp.zeros((128,), jnp.float32).at[:H * HD].set(br1)
    Wl2p = jnp.zeros((HD, 256), jnp.float32).at[:, :H * 2 * HD].set(Wl2)
    Wr2p = jnp.zeros((HD, 256), jnp.float32).at[:, :H * 2 * HD].set(Wr2)
    bl2p = jnp.zeros((256,), jnp.float32).at[:H * 2 * HD].set(bl2)
    br2p = jnp.zeros((256,), jnp.float32).at[:H * 2 * HD].set(br2)

    xl, xr = _proj(xp, Wl1p, bl1p, Wr1p, br1p)
    h1 = _L1(xl, xr, srcs, dsts, att1.reshape(-1), bias1,
             bd10).reshape(NPAD, HD)
    xl, xr = _proj(h1, Wl2p, bl2p, Wr2p, br2p)
    h2 = _L2(xl, xr, srcs, dsts, att2.reshape(-1), bias2,
             bd10).reshape(NPAD, 2 * HD)
    xl, xr = _proj(h2, Wl3, bl3, Wr3, br3)
    h3 = _L3(xl, xr, srcs, dsts, att3.reshape(-1), bias3,
             bd20).reshape(NPAD, 4 * HD)

    gb = jnp.searchsorted(
        batch.astype(jnp.int32), jnp.arange(G + 1, dtype=jnp.int32),
        side="left").astype(jnp.int32)
    gbp = jnp.pad(
        gb[(jnp.arange(NW)[:, None] * 64 + jnp.arange(65)[None, :])],
        ((0, 0), (0, 15)), mode="edge")
    hg = _pool_sc(h3, gbp).reshape(G, 4 * HD)

    mu, logvar, out = _decoder(hg, Wmu, bmu, Wlv, blv, Wd1, bd1, Wd2, bd2,
                               Wd3, bd3)
    x_rec = out[:, :MAXN * 11].reshape(-1, MAXN, 11)
    adj = out[:, MAXN * 11:].reshape(-1, MAXN, MAXN)
    adj_rec = (adj + jnp.transpose(adj, (0, 2, 1))) / 2.0
    return (x_rec, adj_rec, mu, logvar)


# speedup vs baseline: 17.5331x; 1.0284x over previous
import functools

import jax
import jax.numpy as jnp
from jax import lax
from jax.experimental import pallas as pl
from jax.experimental.pallas import tpu as pltpu
from jax.experimental.pallas import tpu_sc as plsc

N = 50000
E = 1600000
F = 11
H = 3
G = 2048
MAXN = 29
HD = 32
LD = 64
OUTDIM = MAXN * F + MAXN * MAXN
OUTPAD = 1280

NW = 32
NPS = 1600
NPAD = NW * NPS
E2 = E + N
EPAD = 512
E2S = E2 + 256

_mesh = plsc.VectorSubcoreMesh(core_axis_name="c", subcore_axis_name="s")


def _hsum(v):
    parts = [v[i] for i in range(16)]
    while len(parts) > 1:
        parts = [parts[i] + parts[i + 1] for i in range(0, len(parts), 2)]
    return parts[0]


def _gat_sc(C, B, CN, K, BDW, HCP):
    HC = H * C

    @functools.partial(
        pl.kernel, mesh=_mesh,
        out_type=jax.ShapeDtypeStruct((NPAD * C,), jnp.float32),
        scratch_types=[
            pltpu.VMEM((B,), jnp.int32),
            pltpu.VMEM((B + 16,), jnp.int32),
            pltpu.VMEM((B, HCP), jnp.float32),
            pltpu.VMEM((B, HCP), jnp.float32),
            pltpu.VMEM((CN * HC,), jnp.float32),
            pltpu.VMEM((CN * 16,), jnp.float32),
            pltpu.VMEM((CN * C,), jnp.float32),
            pltpu.VMEM((HC,), jnp.float32),
            pltpu.VMEM((C,), jnp.float32),
            pltpu.VMEM((BDW,), jnp.int32),
            pltpu.SemaphoreType.DMA,
            pltpu.SemaphoreType.DMA,
        ],
    )
    def ek(xl_hbm, xr_hbm, srcs_hbm, dsts_hbm, att_hbm, bias_hbm, bd_hbm,
           out_hbm, src_v, dst_v, xl_v, xr_v, acc_v, dn_v, outb_v,
           att_v, bias_v, bd_v, sem1, sem2):
        cid = lax.axis_index("c")
        sid = lax.axis_index("s")
        wid = cid * 16 + sid
        pltpu.sync_copy(bd_hbm.at[wid], bd_v)
        pltpu.sync_copy(att_hbm, att_v)
        pltpu.sync_copy(bias_hbm, bias_v)
        iota = lax.iota(jnp.int32, 16)
        zeros16 = jnp.zeros((16,), jnp.float32)

        def chunk(k, kcarry):
            base = wid * NPS + k * CN

            def zb(i, carry):
                acc_v[pl.ds(i * 16, 16)] = zeros16
                return carry

            lax.fori_loop(0, CN * HC // 16, zb, 0)

            def zb2(i, carry):
                dn_v[pl.ds(i * 16, 16)] = zeros16
                return carry

            lax.fori_loop(0, CN, zb2, 0)

            bdw = bd_v[pl.ds(k, 16)]
            lo = bdw[0]
            hi = bdw[1]
            lo8 = (lo // 8) * 8
            nblk = (hi - lo8 + (B - 1)) // B

            def blk(i, carry):
                e0 = lo8 + i * B
                pltpu.sync_copy(srcs_hbm.at[pl.ds(e0, B)], src_v)
                pltpu.sync_copy(dsts_hbm.at[pl.ds(e0, B)],
                                dst_v.at[pl.ds(0, B)])
                cp1 = pltpu.async_copy(xl_hbm.at[src_v], xl_v, sem1)
                cp2 = pltpu.async_copy(
                    xr_hbm.at[dst_v.at[pl.ds(0, B)]], xr_v, sem2)
                cp1.wait()
                cp2.wait()

                def edge(j, ecarry):
                    dr_raw = dst_v[pl.ds(j, 16)][0] - base
                    inb = (dr_raw >= 0) & (dr_raw < CN)
                    mvf = jnp.where(inb, 1.0, 0.0)
                    dr = jnp.clip(dr_raw, 0, CN - 1)
                    dnvec = zeros16
                    for h in range(H):
                        sacc = zeros16
                        xcache = []
                        for c in range(0, C, 16):
                            xlv = xl_v[j, pl.ds(h * C + c, 16)]
                            xrv = xr_v[j, pl.ds(h * C + c, 16)]
                            xcache.append(xlv)
                            v = xlv + xrv
                            lr = jnp.maximum(v, 0.2 * v)
                            sacc = sacc + att_v[pl.ds(h * C + c, 16)] * lr
                        tot = _hsum(sacc)
                        ee = jnp.exp(jnp.full((16,), tot, jnp.float32)) * mvf
                        onehot = (1 - jnp.clip(jnp.abs(iota - h), 0, 1)
                                  ).astype(jnp.float32)
                        dnvec = dnvec + onehot * ee
                        for ci, c in enumerate(range(0, C, 16)):
                            off = dr * HC + h * C + c
                            acc_v[pl.ds(off, 16)] = (
                                acc_v[pl.ds(off, 16)] + xcache[ci] * ee)
                    dn_v[pl.ds(dr * 16, 16)] = (
                        dn_v[pl.ds(dr * 16, 16)] + dnvec)
                    return ecarry

                lax.fori_loop(0, B, edge, 0)
                return carry

            lax.fori_loop(0, nblk, blk, 0)

            def fin(r, carry):
                dnv = dn_v[pl.ds(r * 16, 16)]
                rec = []
                for h in range(H):
                    rec.append(1.0 / (jnp.full((16,), dnv[h]) + 1e-16))
                for c in range(0, C, 16):
                    o = zeros16
                    for h in range(H):
                        o = o + acc_v[pl.ds(r * HC + h * C + c, 16)] * rec[h]
                    o = o * (1.0 / H) + bias_v[pl.ds(c, 16)]
                    o = jnp.maximum(o, 0.0)
                    outb_v[pl.ds(r * C + c, 16)] = o
                return carry

            lax.fori_loop(0, CN, fin, 0)
            pltpu.sync_copy(outb_v, out_hbm.at[pl.ds(base * C, CN * C)])
            return kcarry

        lax.fori_loop(0, K, chunk, 0)

    return ek


@functools.partial(
    pl.kernel, mesh=_mesh,
    out_type=jax.ShapeDtypeStruct((G * 4 * HD,), jnp.float32),
    scratch_types=[
        pltpu.VMEM((80,), jnp.int32),
        pltpu.VMEM((16, 4 * HD), jnp.float32),
        pltpu.VMEM((64 * 4 * HD,), jnp.float32),
    ],
)
def _pool_sc(h_hbm, gb_hbm, hg_hbm, gb_v, rows_v, acc_v):
    cid = lax.axis_index("c")
    sid = lax.axis_index("s")
    wid = cid * 16 + sid
    pltpu.sync_copy(gb_hbm.at[wid], gb_v)
    zeros16 = jnp.zeros((16,), jnp.float32)
    D = 4 * HD

    def zb(i, carry):
        acc_v[pl.ds(i * 16, 16)] = zeros16
        return carry

    lax.fori_loop(0, 64 * D // 16, zb, 0)

    def graph(g, gcarry):
        gw = gb_v[pl.ds(g, 16)]
        lo = gw[0]
        hi = gw[1]
        lo8 = (lo // 8) * 8
        nblk = (hi - lo8 + 15) // 16

        def blk(i, carry):
            e0 = lo8 + i * 16
            pltpu.sync_copy(h_hbm.at[pl.ds(e0, 16)], rows_v)

            def row(r, rcarry):
                ridx = e0 + r
                mf = jnp.where((ridx >= lo) & (ridx < hi), 1.0, 0.0)
                for c in range(0, D, 16):
                    x = rows_v[r, pl.ds(c, 16)]
                    acc_v[pl.ds(g * D + c, 16)] = (
                        acc_v[pl.ds(g * D + c, 16)] + x * mf)
                return rcarry

            lax.fori_loop(0, 16, row, 0)
            return carry

        lax.fori_loop(0, nblk, blk, 0)
        return gcarry

    lax.fori_loop(0, 64, graph, 0)
    pltpu.sync_copy(acc_v, hg_hbm.at[pl.ds(wid * 64 * D, 64 * D)])


def _proj_body(x_ref, wl_ref, bl_ref, wr_ref, br_ref, xl_ref, xr_ref):
    xv = x_ref[...]
    xl_ref[...] = jnp.dot(
        xv, wl_ref[...], preferred_element_type=jnp.float32) + bl_ref[...]
    xr_ref[...] = jnp.dot(
        xv, wr_ref[...], preferred_element_type=jnp.float32) + br_ref[...]


def _proj(x, Wl, bl, Wr, br):
    cin = x.shape[1]
    hc = Wl.shape[1]
    rb = 512
    full = lambda *s: pl.BlockSpec(s, lambda i: (0,) * len(s))
    return pl.pallas_call(
        _proj_body,
        grid=(NPAD // rb,),
        in_specs=[
            pl.BlockSpec((rb, cin), lambda i: (i, 0)),
            full(cin, hc), full(1, hc), full(cin, hc), full(1, hc),
        ],
        out_specs=[
            pl.BlockSpec((rb, hc), lambda i: (i, 0)),
            pl.BlockSpec((rb, hc), lambda i: (i, 0)),
        ],
        out_shape=[
            jax.ShapeDtypeStruct((NPAD, hc), jnp.float32),
            jax.ShapeDtypeStruct((NPAD, hc), jnp.float32),
        ],
    )(x, Wl, bl.reshape(1, hc), Wr, br.reshape(1, hc))


def _decoder_body(hg_ref, wmu_ref, bmu_ref, wlv_ref, blv_ref, wd1_ref, bd1_ref,
                  wd2_ref, bd2_ref, wd3_ref, bd3_ref, mu_ref, lv_ref, out_ref):
    hg = hg_ref[...]
    mu = hg @ wmu_ref[...] + bmu_ref[...]
    mu_ref[...] = mu
    lv_ref[...] = hg @ wlv_ref[...] + blv_ref[...]
    o = jnp.maximum(mu @ wd1_ref[...] + bd1_ref[...], 0.0)
    o = jnp.maximum(o @ wd2_ref[...] + bd2_ref[...], 0.0)
    out_ref[...] = o @ wd3_ref[...] + bd3_ref[...]


def _decoder(hg, Wmu, bmu, Wlv, blv, Wd1, bd1, Wd2, bd2, Wd3, bd3):
    gb = 256
    Wd3p = jnp.zeros((8 * HD, OUTPAD), jnp.float32).at[:, :OUTDIM].set(Wd3)
    bd3p = jnp.zeros((1, OUTPAD), jnp.float32).at[0, :OUTDIM].set(bd3)
    full = lambda *s: pl.BlockSpec(s, lambda i: (0,) * len(s))
    mu, lv, out = pl.pallas_call(
        _decoder_body,
        grid=(G // gb,),
        in_specs=[
            pl.BlockSpec((gb, 4 * HD), lambda i: (i, 0)),
            full(4 * HD, LD), full(1, LD), full(4 * HD, LD), full(1, LD),
            full(LD, 4 * HD), full(1, 4 * HD), full(4 * HD, 8 * HD),
            full(1, 8 * HD), full(8 * HD, OUTPAD), full(1, OUTPAD),
        ],
        out_specs=[
            pl.BlockSpec((gb, LD), lambda i: (i, 0)),
            pl.BlockSpec((gb, LD), lambda i: (i, 0)),
            pl.BlockSpec((gb, OUTPAD), lambda i: (i, 0)),
        ],
        out_shape=[
            jax.ShapeDtypeStruct((G, LD), jnp.float32),
            jax.ShapeDtypeStruct((G, LD), jnp.float32),
            jax.ShapeDtypeStruct((G, OUTPAD), jnp.float32),
        ],
    )(hg, Wmu, bmu.reshape(1, LD), Wlv, blv.reshape(1, LD), Wd1,
      bd1.reshape(1, 4 * HD), Wd2, bd2.reshape(1, 8 * HD), Wd3p, bd3p)
    return mu, lv, out[:, :OUTDIM]


_L1 = _gat_sc(C=HD, B=128, CN=160, K=10, BDW=32, HCP=128)
_L2 = _gat_sc(C=2 * HD, B=128, CN=160, K=10, BDW=32, HCP=256)
_L3 = _gat_sc(C=4 * HD, B=96, CN=80, K=20, BDW=40, HCP=384)


def _bd_rows(dsts_view, cn, k):
    q = jnp.arange(NW * k + 1, dtype=jnp.int32) * cn
    bd = jnp.searchsorted(dsts_view, q, side="left").astype(jnp.int32)
    rows = bd[(jnp.arange(NW)[:, None] * k + jnp.arange(k + 1)[None, :])]
    padw = 32 if k == 10 else 40
    return jnp.pad(rows, ((0, 0), (0, padw - (k + 1))), mode="edge")


def kernel(x, edge_index, batch, Wl1, bl1, Wr1, br1, att1, bias1, Wl2, bl2,
           Wr2, br2, att2, bias2, Wl3, bl3, Wr3, br3, att3, bias3, Wmu, bmu,
           Wlv, blv, Wd1, bd1, Wd2, bd2, Wd3, bd3):
    loop = jnp.arange(N, dtype=jnp.int32)
    src0 = jnp.concatenate([edge_index[0].astype(jnp.int32), loop])
    dst0 = jnp.concatenate([edge_index[1].astype(jnp.int32), loop])
    dsts, srcs = lax.sort_key_val(dst0, src0)
    srcs = jnp.concatenate([srcs, jnp.zeros((EPAD,), jnp.int32)])
    dsts = jnp.concatenate(
        [dsts, jnp.full((EPAD,), NPAD - 8, jnp.int32)])

    bd10 = _bd_rows(dsts[:E2S], 160, 10)
    bd20 = _bd_rows(dsts[:E2S], 80, 20)

    xp = jnp.zeros((NPAD, 16), jnp.float32).at[:N, :F].set(x)
    Wl1p = jnp.zeros((16, 128), jnp.float32).at[:F, :H * HD].set(Wl1)
    Wr1p = jnp.zeros((16, 128), jnp.float32).at[:F, :H * HD].set(Wr1)
    bl1p = jnp.zeros((128,), jnp.float32).at[:H * HD].set(bl1)
    br1p = jnp.zeros((128,), jnp.float32).at[:H * HD].set(br1)
    Wl2p = jnp.zeros((HD, 256), jnp.float32).at[:, :H * 2 * HD].set(Wl2)
    Wr2p = jnp.zeros((HD, 256), jnp.float32).at[:, :H * 2 * HD].set(Wr2)
    bl2p = jnp.zeros((256,), jnp.float32).at[:H * 2 * HD].set(bl2)
    br2p = jnp.zeros((256,), jnp.float32).at[:H * 2 * HD].set(br2)

    xl, xr = _proj(xp, Wl1p, bl1p, Wr1p, br1p)
    h1 = _L1(xl, xr, srcs, dsts, att1.reshape(-1), bias1,
             bd10).reshape(NPAD, HD)
    xl, xr = _proj(h1, Wl2p, bl2p, Wr2p, br2p)
    h2 = _L2(xl, xr, srcs, dsts, att2.reshape(-1), bias2,
             bd10).reshape(NPAD, 2 * HD)
    xl, xr = _proj(h2, Wl3, bl3, Wr3, br3)
    h3 = _L3(xl, xr, srcs, dsts, att3.reshape(-1), bias3,
             bd20).reshape(NPAD, 4 * HD)

    gb = jnp.searchsorted(
        batch.astype(jnp.int32), jnp.arange(G + 1, dtype=jnp.int32),
        side="left").astype(jnp.int32)
    gbp = jnp.pad(
        gb[(jnp.arange(NW)[:, None] * 64 + jnp.arange(65)[None, :])],
        ((0, 0), (0, 15)), mode="edge")
    hg = _pool_sc(h3, gbp).reshape(G, 4 * HD)

    mu, logvar, out = _decoder(hg, Wmu, bmu, Wlv, blv, Wd1, bd1, Wd2, bd2,
                               Wd3, bd3)
    x_rec = out[:, :MAXN * 11].reshape(-1, MAXN, 11)
    adj = out[:, MAXN * 11:].reshape(-1, MAXN, MAXN)
    adj_rec = (adj + jnp.transpose(adj, (0, 2, 1))) / 2.0
    return (x_rec, adj_rec, mu, logvar)


# speedup vs baseline: 18.0351x; 1.0286x over previous
import functools

import jax
import jax.numpy as jnp
from jax import lax
from jax.experimental import pallas as pl
from jax.experimental.pallas import tpu as pltpu
from jax.experimental.pallas import tpu_sc as plsc

N = 50000
E = 1600000
F = 11
H = 3
G = 2048
MAXN = 29
HD = 32
LD = 64
OUTDIM = MAXN * F + MAXN * MAXN
OUTPAD = 1280

NW = 32
NPS = 1600
NPAD = NW * NPS
E2 = E + N
EPAD = 512
E2S = E2 + 256

_mesh = plsc.VectorSubcoreMesh(core_axis_name="c", subcore_axis_name="s")


def _hsum(v):
    parts = [v[i] for i in range(16)]
    while len(parts) > 1:
        parts = [parts[i] + parts[i + 1] for i in range(0, len(parts), 2)]
    return parts[0]


def _gat_sc(C, B, CN, K, BDW, HCP):
    HC = H * C

    @functools.partial(
        pl.kernel, mesh=_mesh,
        out_type=jax.ShapeDtypeStruct((NPAD * C,), jnp.float32),
        scratch_types=[
            pltpu.VMEM((B,), jnp.int32),
            pltpu.VMEM((B + 16,), jnp.int32),
            pltpu.VMEM((B, HCP), jnp.float32),
            pltpu.VMEM((B, HCP), jnp.float32),
            pltpu.VMEM((CN * HC,), jnp.float32),
            pltpu.VMEM((CN * 16,), jnp.float32),
            pltpu.VMEM((CN * C,), jnp.float32),
            pltpu.VMEM((32,), jnp.float32),
            pltpu.VMEM((HC,), jnp.float32),
            pltpu.VMEM((C,), jnp.float32),
            pltpu.VMEM((BDW,), jnp.int32),
            pltpu.SemaphoreType.DMA,
            pltpu.SemaphoreType.DMA,
        ],
    )
    def ek(xl_hbm, xr_hbm, srcs_hbm, dsts_hbm, att_hbm, bias_hbm, bd_hbm,
           out_hbm, src_v, dst_v, xl_v, xr_v, acc_v, dn_v, outb_v,
           tmp_v, att_v, bias_v, bd_v, sem1, sem2):
        cid = lax.axis_index("c")
        sid = lax.axis_index("s")
        wid = cid * 16 + sid
        pltpu.sync_copy(bd_hbm.at[wid], bd_v)
        pltpu.sync_copy(att_hbm, att_v)
        pltpu.sync_copy(bias_hbm, bias_v)
        iota = lax.iota(jnp.int32, 16)
        zeros16 = jnp.zeros((16,), jnp.float32)
        tmp_v[pl.ds(0, 16)] = zeros16
        tmp_v[pl.ds(16, 16)] = zeros16

        def chunk(k, kcarry):
            base = wid * NPS + k * CN

            def zb(i, carry):
                acc_v[pl.ds(i * 16, 16)] = zeros16
                return carry

            lax.fori_loop(0, CN * HC // 16, zb, 0)

            def zb2(i, carry):
                dn_v[pl.ds(i * 16, 16)] = zeros16
                return carry

            lax.fori_loop(0, CN, zb2, 0)

            bdw = bd_v[pl.ds(k, 16)]
            lo = bdw[0]
            hi = bdw[1]
            lo8 = (lo // 8) * 8
            nblk = (hi - lo8 + (B - 1)) // B

            def blk(i, carry):
                e0 = lo8 + i * B
                pltpu.sync_copy(srcs_hbm.at[pl.ds(e0, B)], src_v)
                pltpu.sync_copy(dsts_hbm.at[pl.ds(e0, B)],
                                dst_v.at[pl.ds(0, B)])
                cp1 = pltpu.async_copy(xl_hbm.at[src_v], xl_v, sem1)
                cp2 = pltpu.async_copy(
                    xr_hbm.at[dst_v.at[pl.ds(0, B)]], xr_v, sem2)
                cp1.wait()
                cp2.wait()

                def edge(j, ecarry):
                    dr_raw = dst_v[pl.ds(j, 16)][0] - base
                    inb = (dr_raw >= 0) & (dr_raw < CN)
                    mvf = jnp.where(inb, 1.0, 0.0)
                    dr = jnp.clip(dr_raw, 0, CN - 1)
                    dnvec = zeros16
                    for h in range(H):
                        sacc = zeros16
                        xcache = []
                        for c in range(0, C, 16):
                            xlv = xl_v[j, pl.ds(h * C + c, 16)]
                            xrv = xr_v[j, pl.ds(h * C + c, 16)]
                            xcache.append(xlv)
                            v = xlv + xrv
                            lr = jnp.maximum(v, 0.2 * v)
                            sacc = sacc + att_v[pl.ds(h * C + c, 16)] * lr
                        tmp_v[pl.ds(0, 16)] = sacc
                        a = (tmp_v[pl.ds(0, 16)] + tmp_v[pl.ds(4, 16)]
                             + tmp_v[pl.ds(8, 16)] + tmp_v[pl.ds(12, 16)])
                        tot = (a[0] + a[1]) + (a[2] + a[3])
                        ee = jnp.exp(jnp.full((16,), tot, jnp.float32)) * mvf
                        onehot = (1 - jnp.clip(jnp.abs(iota - h), 0, 1)
                                  ).astype(jnp.float32)
                        dnvec = dnvec + onehot * ee
                        for ci, c in enumerate(range(0, C, 16)):
                            off = dr * HC + h * C + c
                            acc_v[pl.ds(off, 16)] = (
                                acc_v[pl.ds(off, 16)] + xcache[ci] * ee)
                    dn_v[pl.ds(dr * 16, 16)] = (
                        dn_v[pl.ds(dr * 16, 16)] + dnvec)
                    return ecarry

                lax.fori_loop(0, B, edge, 0)
                return carry

            lax.fori_loop(0, nblk, blk, 0)

            def fin(r, carry):
                dnv = dn_v[pl.ds(r * 16, 16)]
                rec = []
                for h in range(H):
                    rec.append(1.0 / (jnp.full((16,), dnv[h]) + 1e-16))
                for c in range(0, C, 16):
                    o = zeros16
                    for h in range(H):
                        o = o + acc_v[pl.ds(r * HC + h * C + c, 16)] * rec[h]
                    o = o * (1.0 / H) + bias_v[pl.ds(c, 16)]
                    o = jnp.maximum(o, 0.0)
                    outb_v[pl.ds(r * C + c, 16)] = o
                return carry

            lax.fori_loop(0, CN, fin, 0)
            pltpu.sync_copy(outb_v, out_hbm.at[pl.ds(base * C, CN * C)])
            return kcarry

        lax.fori_loop(0, K, chunk, 0)

    return ek


@functools.partial(
    pl.kernel, mesh=_mesh,
    out_type=jax.ShapeDtypeStruct((G * 4 * HD,), jnp.float32),
    scratch_types=[
        pltpu.VMEM((80,), jnp.int32),
        pltpu.VMEM((16, 4 * HD), jnp.float32),
        pltpu.VMEM((64 * 4 * HD,), jnp.float32),
    ],
)
def _pool_sc(h_hbm, gb_hbm, hg_hbm, gb_v, rows_v, acc_v):
    cid = lax.axis_index("c")
    sid = lax.axis_index("s")
    wid = cid * 16 + sid
    pltpu.sync_copy(gb_hbm.at[wid], gb_v)
    zeros16 = jnp.zeros((16,), jnp.float32)
    D = 4 * HD

    def zb(i, carry):
        acc_v[pl.ds(i * 16, 16)] = zeros16
        return carry

    lax.fori_loop(0, 64 * D // 16, zb, 0)

    def graph(g, gcarry):
        gw = gb_v[pl.ds(g, 16)]
        lo = gw[0]
        hi = gw[1]
        lo8 = (lo // 8) * 8
        nblk = (hi - lo8 + 15) // 16

        def blk(i, carry):
            e0 = lo8 + i * 16
            pltpu.sync_copy(h_hbm.at[pl.ds(e0, 16)], rows_v)

            def row(r, rcarry):
                ridx = e0 + r
                mf = jnp.where((ridx >= lo) & (ridx < hi), 1.0, 0.0)
                for c in range(0, D, 16):
                    x = rows_v[r, pl.ds(c, 16)]
                    acc_v[pl.ds(g * D + c, 16)] = (
                        acc_v[pl.ds(g * D + c, 16)] + x * mf)
                return rcarry

            lax.fori_loop(0, 16, row, 0)
            return carry

        lax.fori_loop(0, nblk, blk, 0)
        return gcarry

    lax.fori_loop(0, 64, graph, 0)
    pltpu.sync_copy(acc_v, hg_hbm.at[pl.ds(wid * 64 * D, 64 * D)])


def _proj_body(x_ref, wl_ref, bl_ref, wr_ref, br_ref, xl_ref, xr_ref):
    xv = x_ref[...]
    xl_ref[...] = jnp.dot(
        xv, wl_ref[...], preferred_element_type=jnp.float32) + bl_ref[...]
    xr_ref[...] = jnp.dot(
        xv, wr_ref[...], preferred_element_type=jnp.float32) + br_ref[...]


def _proj(x, Wl, bl, Wr, br):
    cin = x.shape[1]
    hc = Wl.shape[1]
    rb = 512
    full = lambda *s: pl.BlockSpec(s, lambda i: (0,) * len(s))
    return pl.pallas_call(
        _proj_body,
        grid=(NPAD // rb,),
        in_specs=[
            pl.BlockSpec((rb, cin), lambda i: (i, 0)),
            full(cin, hc), full(1, hc), full(cin, hc), full(1, hc),
        ],
        out_specs=[
            pl.BlockSpec((rb, hc), lambda i: (i, 0)),
            pl.BlockSpec((rb, hc), lambda i: (i, 0)),
        ],
        out_shape=[
            jax.ShapeDtypeStruct((NPAD, hc), jnp.float32),
            jax.ShapeDtypeStruct((NPAD, hc), jnp.float32),
        ],
    )(x, Wl, bl.reshape(1, hc), Wr, br.reshape(1, hc))


def _decoder_body(hg_ref, wmu_ref, bmu_ref, wlv_ref, blv_ref, wd1_ref, bd1_ref,
                  wd2_ref, bd2_ref, wd3_ref, bd3_ref, mu_ref, lv_ref, out_ref):
    hg = hg_ref[...]
    mu = hg @ wmu_ref[...] + bmu_ref[...]
    mu_ref[...] = mu
    lv_ref[...] = hg @ wlv_ref[...] + blv_ref[...]
    o = jnp.maximum(mu @ wd1_ref[...] + bd1_ref[...], 0.0)
    o = jnp.maximum(o @ wd2_ref[...] + bd2_ref[...], 0.0)
    out_ref[...] = o @ wd3_ref[...] + bd3_ref[...]


def _decoder(hg, Wmu, bmu, Wlv, blv, Wd1, bd1, Wd2, bd2, Wd3, bd3):
    gb = 256
    Wd3p = jnp.zeros((8 * HD, OUTPAD), jnp.float32).at[:, :OUTDIM].set(Wd3)
    bd3p = jnp.zeros((1, OUTPAD), jnp.float32).at[0, :OUTDIM].set(bd3)
    full = lambda *s: pl.BlockSpec(s, lambda i: (0,) * len(s))
    mu, lv, out = pl.pallas_call(
        _decoder_body,
        grid=(G // gb,),
        in_specs=[
            pl.BlockSpec((gb, 4 * HD), lambda i: (i, 0)),
            full(4 * HD, LD), full(1, LD), full(4 * HD, LD), full(1, LD),
            full(LD, 4 * HD), full(1, 4 * HD), full(4 * HD, 8 * HD),
            full(1, 8 * HD), full(8 * HD, OUTPAD), full(1, OUTPAD),
        ],
        out_specs=[
            pl.BlockSpec((gb, LD), lambda i: (i, 0)),
            pl.BlockSpec((gb, LD), lambda i: (i, 0)),
            pl.BlockSpec((gb, OUTPAD), lambda i: (i, 0)),
        ],
        out_shape=[
            jax.ShapeDtypeStruct((G, LD), jnp.float32),
            jax.ShapeDtypeStruct((G, LD), jnp.float32),
            jax.ShapeDtypeStruct((G, OUTPAD), jnp.float32),
        ],
    )(hg, Wmu, bmu.reshape(1, LD), Wlv, blv.reshape(1, LD), Wd1,
      bd1.reshape(1, 4 * HD), Wd2, bd2.reshape(1, 8 * HD), Wd3p, bd3p)
    return mu, lv, out[:, :OUTDIM]


_L1 = _gat_sc(C=HD, B=128, CN=160, K=10, BDW=32, HCP=128)
_L2 = _gat_sc(C=2 * HD, B=128, CN=160, K=10, BDW=32, HCP=256)
_L3 = _gat_sc(C=4 * HD, B=96, CN=80, K=20, BDW=40, HCP=384)


def _bd_rows(dsts_view, cn, k):
    q = jnp.arange(NW * k + 1, dtype=jnp.int32) * cn
    bd = jnp.searchsorted(dsts_view, q, side="left").astype(jnp.int32)
    rows = bd[(jnp.arange(NW)[:, None] * k + jnp.arange(k + 1)[None, :])]
    padw = 32 if k == 10 else 40
    return jnp.pad(rows, ((0, 0), (0, padw - (k + 1))), mode="edge")


def kernel(x, edge_index, batch, Wl1, bl1, Wr1, br1, att1, bias1, Wl2, bl2,
           Wr2, br2, att2, bias2, Wl3, bl3, Wr3, br3, att3, bias3, Wmu, bmu,
           Wlv, blv, Wd1, bd1, Wd2, bd2, Wd3, bd3):
    loop = jnp.arange(N, dtype=jnp.int32)
    src0 = jnp.concatenate([edge_index[0].astype(jnp.int32), loop])
    dst0 = jnp.concatenate([edge_index[1].astype(jnp.int32), loop])
    dsts, srcs = lax.sort_key_val(dst0, src0)
    srcs = jnp.concatenate([srcs, jnp.zeros((EPAD,), jnp.int32)])
    dsts = jnp.concatenate(
        [dsts, jnp.full((EPAD,), NPAD - 8, jnp.int32)])

    bd10 = _bd_rows(dsts[:E2S], 160, 10)
    bd20 = _bd_rows(dsts[:E2S], 80, 20)

    xp = jnp.zeros((NPAD, 16), jnp.float32).at[:N, :F].set(x)
    Wl1p = jnp.zeros((16, 128), jnp.float32).at[:F, :H * HD].set(Wl1)
    Wr1p = jnp.zeros((16, 128), jnp.float32).at[:F, :H * HD].set(Wr1)
    bl1p = jnp.zeros((128,), jnp.float32).at[:H * HD].set(bl1)
    br1p = jnp.zeros((128,), jnp.float32).at[:H * HD].set(br1)
    Wl2p = jnp.zeros((HD, 256), jnp.float32).at[:, :H * 2 * HD].set(Wl2)
    Wr2p = jnp.zeros((HD, 256), jnp.float32).at[:, :H * 2 * HD].set(Wr2)
    bl2p = jnp.zeros((256,), jnp.float32).at[:H * 2 * HD].set(bl2)
    br2p = jnp.zeros((256,), jnp.float32).at[:H * 2 * HD].set(br2)

    xl, xr = _proj(xp, Wl1p, bl1p, Wr1p, br1p)
    h1 = _L1(xl, xr, srcs, dsts, att1.reshape(-1), bias1,
             bd10).reshape(NPAD, HD)
    xl, xr = _proj(h1, Wl2p, bl2p, Wr2p, br2p)
    h2 = _L2(xl, xr, srcs, dsts, att2.reshape(-1), bias2,
             bd10).reshape(NPAD, 2 * HD)
    xl, xr = _proj(h2, Wl3, bl3, Wr3, br3)
    h3 = _L3(xl, xr, srcs, dsts, att3.reshape(-1), bias3,
             bd20).reshape(NPAD, 4 * HD)

    gb = jnp.searchsorted(
        batch.astype(jnp.int32), jnp.arange(G + 1, dtype=jnp.int32),
        side="left").astype(jnp.int32)
    gbp = jnp.pad(
        gb[(jnp.arange(NW)[:, None] * 64 + jnp.arange(65)[None, :])],
        ((0, 0), (0, 15)), mode="edge")
    hg = _pool_sc(h3, gbp).reshape(G, 4 * HD)

    mu, logvar, out = _decoder(hg, Wmu, bmu, Wlv, blv, Wd1, bd1, Wd2, bd2,
                               Wd3, bd3)
    x_rec = out[:, :MAXN * 11].reshape(-1, MAXN, 11)
    adj = out[:, MAXN * 11:].reshape(-1, MAXN, MAXN)
    adj_rec = (adj + jnp.transpose(adj, (0, 2, 1))) / 2.0
    return (x_rec, adj_rec, mu, logvar)
